# baseline (speedup 1.0000x reference)
import functools

import jax
import jax.numpy as jnp
from jax import lax
from jax.experimental import pallas as pl
from jax.experimental.pallas import tpu as pltpu
from jax.experimental.pallas import tpu_sc as plsc

B = 16384
D = 64
NC = 2
NS = 16
NW = NC * NS
BPW = B // NW
L = 16

_mesh = plsc.VectorSubcoreMesh(core_axis_name="c", subcore_axis_name="s")


def _neg_sqrt(x):
    i = plsc.bitcast(x, jnp.int32)
    i = jnp.int32(0x5F3759DF) - lax.shift_right_logical(i, 1)
    y = plsc.bitcast(i, jnp.float32)
    for _ in range(3):
        y = y * (1.5 - 0.5 * x * y * y)
    return jnp.where(x > 0.0, -x * y, 0.0)


@functools.partial(
    pl.kernel,
    mesh=_mesh,
    compiler_params=pltpu.CompilerParams(
        needs_layout_passes=False, use_tc_tiling_on_sc=False),
    out_type=jax.ShapeDtypeStruct((B,), jnp.float32),
    scratch_types=[
        pltpu.VMEM((BPW,), jnp.int32),
        pltpu.VMEM((BPW,), jnp.int32),
        pltpu.VMEM((BPW,), jnp.int32),
        pltpu.VMEM((BPW, D), jnp.bfloat16),
        pltpu.VMEM((BPW, D), jnp.bfloat16),
        pltpu.VMEM((BPW, D), jnp.bfloat16),
        pltpu.VMEM((BPW,), jnp.float32),
        pltpu.SemaphoreType.DMA,
        pltpu.SemaphoreType.DMA,
        pltpu.SemaphoreType.DMA,
    ],
)
def _transe_sc(ent_hbm, rel_hbm, heads_hbm, rels_hbm, tails_hbm, out_hbm,
               hidx, ridx, tidx, hrow, rrow, trow, outv,
               sem_h, sem_r, sem_t):
    wid = lax.axis_index("s") * NC + lax.axis_index("c")
    base = wid * BPW

    pltpu.sync_copy(heads_hbm.at[pl.ds(base, BPW)], hidx)
    pltpu.sync_copy(rels_hbm.at[pl.ds(base, BPW)], ridx)
    pltpu.sync_copy(tails_hbm.at[pl.ds(base, BPW)], tidx)

    cp_h = pltpu.async_copy(ent_hbm.at[hidx], hrow, sem_h)
    cp_r = pltpu.async_copy(rel_hbm.at[ridx], rrow, sem_r)
    cp_t = pltpu.async_copy(ent_hbm.at[tidx], trow, sem_t)
    cp_h.wait()
    cp_r.wait()
    cp_t.wait()

    lanes = lax.iota(jnp.int32, L)

    def body(g, carry):
        packed = jnp.zeros((L,), jnp.float32)
        for j in range(L):
            item = g * L + j
            acc = jnp.zeros((L,), jnp.float32)
            for c in range(D // 32):
                sl = pl.ds(c * 32, 32)
                h0, h1 = plsc.unpack(hrow[item, sl],
                                     format=plsc.PackFormat.INTERLEAVED)
                r0, r1 = plsc.unpack(rrow[item, sl],
                                     format=plsc.PackFormat.INTERLEAVED)
                t0, t1 = plsc.unpack(trow[item, sl],
                                     format=plsc.PackFormat.INTERLEAVED)
                d0 = h0 + r0 - t0
                d1 = h1 + r1 - t1
                acc = acc + d0 * d0 + d1 * d1
            packed = jnp.where(lanes == j, jnp.sum(acc), packed)
        outv[pl.ds(g * L, L)] = _neg_sqrt(packed)
        return carry

    lax.fori_loop(0, BPW // L, body, 0)
    pltpu.sync_copy(outv, out_hbm.at[pl.ds(base, BPW)])


def kernel(entity_emb, relation_emb, heads, relations, tails):
    return _transe_sc(entity_emb.astype(jnp.bfloat16),
                      relation_emb.astype(jnp.bfloat16),
                      heads.astype(jnp.int32),
                      relations.astype(jnp.int32),
                      tails.astype(jnp.int32))


# speedup vs baseline: 3.0848x; 3.0848x over previous
import functools

import jax
import jax.numpy as jnp
from jax import lax
from jax.experimental import pallas as pl
from jax.experimental.pallas import tpu as pltpu
from jax.experimental.pallas import tpu_sc as plsc

B = 16384
D = 64
E = 1000000
NC = 2
NS = 16
NW = NC * NS
BPW = B // NW
L = 16
CH = 128
NCOLS = E // 128
KFULL = NCOLS // NW

_mesh = plsc.VectorSubcoreMesh(core_axis_name="c", subcore_axis_name="s")


def _neg_sqrt(x):
    i = plsc.bitcast(x, jnp.int32)
    i = jnp.int32(0x5F3759DF) - lax.shift_right_logical(i, 1)
    y = plsc.bitcast(i, jnp.float32)
    for _ in range(3):
        y = y * (1.5 - 0.5 * x * y * y)
    return jnp.where(x > 0.0, -x * y, 0.0)


def _splat(v):
    return jnp.full((L,), v, jnp.int32)


@functools.partial(
    pl.kernel,
    mesh=_mesh,
    compiler_params=pltpu.CompilerParams(
        needs_layout_passes=False, use_tc_tiling_on_sc=True),
    out_type=jax.ShapeDtypeStruct((2 * B * D,), jnp.float32),
    scratch_types=[
        pltpu.VMEM((B,), jnp.int32),
        pltpu.VMEM((B,), jnp.int32),
        pltpu.VMEM((2 * B,), jnp.int32),
        pltpu.VMEM((2 * B,), jnp.int32),
        pltpu.VMEM((D, 128), jnp.float32),
        pltpu.VMEM((D, 128), jnp.float32),
        pltpu.VMEM((8 * D,), jnp.float32),
        pltpu.VMEM((D, 64), jnp.float32),
        pltpu.SMEM((1,), jnp.int32),
        pltpu.SemaphoreType.DMA,
        pltpu.SemaphoreType.DMA,
        pltpu.SemaphoreType.DMA,
    ],
)
def _scan_extract(entT_hbm, heads_hbm, tails_hbm, stage_hbm,
                  hall, tall, mA, mB, bufA, bufB, ring, bufP, ptr_ref,
                  semA, semB, sem_st):
    w = lax.axis_index("s") * NC + lax.axis_index("c")
    iota = lax.iota(jnp.int32, L)

    pltpu.sync_copy(heads_hbm, hall)
    pltpu.sync_copy(tails_hbm, tall)

    def match(src_ref, tag):
        def step(g, off):
            ev = src_ref[pl.ds(g * L, L)]
            tc = lax.shift_right_logical(ev, 7)
            m = (tc & 31) == w
            slots = g * L + iota + tag
            plsc.store_compressed(mA.at[pl.ds(off, L)], slots, mask=m)
            return off + plsc.all_reduce_population_count(m)[0]
        return step

    off = lax.fori_loop(0, B // L, match(hall, 0), jnp.int32(0))
    n = lax.fori_loop(0, B // L, match(tall, B), off)

    def _lc_of(svec):
        slot = svec & (B - 1)
        ist = svec >= B
        eh = plsc.load_gather(hall, [slot])
        et = plsc.load_gather(tall, [slot])
        ev = jnp.where(ist, et, eh)
        tc = lax.shift_right_logical(ev, 7)
        return lax.shift_right_logical(tc - w, 5)

    nv = lax.shift_right_logical(n + L - 1, 4)

    def radix(src, dst, shift):
        def hist_step(g, hist):
            svec = src[pl.ds(g * L, L)]
            valid = (g * L + iota) < n
            dig = lax.shift_right_logical(_lc_of(svec), shift) & 15
            for d in range(16):
                cnt = plsc.all_reduce_population_count(valid & (dig == d))
                hist = hist + jnp.where(iota == d, cnt, 0)
            return hist

        hist = lax.fori_loop(0, nv, hist_step, jnp.zeros((L,), jnp.int32))
        excl = plsc.cumsum(hist) - hist

        def scat_step(g, offs):
            svec = src[pl.ds(g * L, L)]
            valid = (g * L + iota) < n
            dig = lax.shift_right_logical(_lc_of(svec), shift) & 15
            new = []
            for d in range(16):
                m = valid & (dig == d)
                plsc.store_compressed(dst.at[pl.ds(offs[d], L)], svec, mask=m)
                new.append(offs[d] + plsc.all_reduce_population_count(m)[0])
            return tuple(new)

        lax.fori_loop(0, nv, scat_step, tuple(excl[d] for d in range(16)))

    radix(mA, mB, 0)
    radix(mB, mA, 4)

    ptr_ref[0] = 0

    def process(buf, k):
        def cond(p):
            pc = jnp.minimum(p, jnp.maximum(n - 1, 0))
            lc = _lc_of(plsc.load_gather(mA, [_splat(pc)]))[0]
            return (p < n) & (lc == k)

        def body(p):
            svec = plsc.load_gather(mA, [_splat(p)])
            slot = svec & (B - 1)
            eh = plsc.load_gather(hall, [slot])
            et = plsc.load_gather(tall, [slot])
            ev = jnp.where(svec >= B, et, eh)
            lane = (ev & 127)[0]
            rs = p & 7
            pl.when(p >= 8)(lambda: pltpu.make_async_copy(
                ring.at[pl.ds(0, D)], stage_hbm.at[pl.ds(0, D)],
                sem_st).wait())
            for kk in range(D // L):
                v = plsc.load_gather(buf, [kk * L + iota, _splat(lane)])
                ring[pl.ds(rs * D + kk * L, L)] = v
            pltpu.async_copy(ring.at[pl.ds(rs * D, D)],
                             stage_hbm.at[pl.ds(svec[0] * D, D)], sem_st)
            return p + 1

        ptr_ref[0] = lax.while_loop(cond, body, ptr_ref[0])

    def col_off(k):
        return pl.multiple_of((w + NW * k) * 128, 128)

    pltpu.async_copy(entT_hbm.at[:, pl.ds(col_off(0), 128)], bufA, semA)

    def pair(i, carry):
        pltpu.async_copy(entT_hbm.at[:, pl.ds(col_off(2 * i + 1), 128)],
                         bufB, semB)
        pltpu.make_async_copy(entT_hbm.at[:, pl.ds(0, 128)], bufA, semA).wait()
        process(bufA, 2 * i)
        @pl.when(i < KFULL // 2 - 1)
        def _prefetch_a():
            pltpu.async_copy(
                entT_hbm.at[:, pl.ds(col_off(2 * i + 2), 128)], bufA, semA)
        pltpu.make_async_copy(entT_hbm.at[:, pl.ds(0, 128)], bufB, semB).wait()
        process(bufB, 2 * i + 1)
        return carry

    lax.fori_loop(0, KFULL // 2, pair, 0)

    @pl.when(w <= 3)
    def _tail_full():
        pltpu.sync_copy(entT_hbm.at[:, pl.ds(col_off(KFULL), 128)], bufA)
        process(bufA, KFULL)

    @pl.when(w == 4)
    def _tail_partial():
        pltpu.sync_copy(entT_hbm.at[:, pl.ds(NCOLS * 128, 64)], bufP)
        process(bufP, KFULL)

    def drain(j, carry):
        pl.when(j < jnp.minimum(n, 8))(lambda: pltpu.make_async_copy(
            ring.at[pl.ds(0, D)], stage_hbm.at[pl.ds(0, D)],
            sem_st).wait())
        return carry

    lax.fori_loop(0, 8, drain, 0)


@functools.partial(
    pl.kernel,
    mesh=_mesh,
    compiler_params=pltpu.CompilerParams(
        needs_layout_passes=False, use_tc_tiling_on_sc=True),
    out_type=jax.ShapeDtypeStruct((B,), jnp.float32),
    scratch_types=[
        pltpu.VMEM((BPW,), jnp.int32),
        pltpu.VMEM((CH * D,), jnp.float32),
        pltpu.VMEM((CH, D), jnp.float32),
        pltpu.VMEM((CH * D,), jnp.float32),
        pltpu.VMEM((BPW,), jnp.float32),
        pltpu.SemaphoreType.DMA,
        pltpu.SemaphoreType.DMA,
        pltpu.SemaphoreType.DMA,
    ],
)
def _compute(stage_hbm, rel_hbm, rels_hbm, out_hbm,
             ridx, hrow, rrow, trow, outv, sem_h, sem_r, sem_t):
    wid = lax.axis_index("s") * NC + lax.axis_index("c")
    base = wid * BPW
    lanes = lax.iota(jnp.int32, L)

    pltpu.sync_copy(rels_hbm.at[pl.ds(base, BPW)], ridx)

    def chunk(ci, carry):
        pltpu.async_copy(stage_hbm.at[pl.ds((base + ci * CH) * D, CH * D)],
                         hrow, sem_h)
        pltpu.async_copy(
            stage_hbm.at[pl.ds((B + base + ci * CH) * D, CH * D)],
            trow, sem_t)

        def fetch(g, c2):
            rv = ridx[pl.ds(ci * CH + g * L, L)]
            for j in range(L):
                pltpu.async_copy(rel_hbm.at[rv[j]], rrow.at[g * L + j], sem_r)
            return c2

        lax.fori_loop(0, CH // L, fetch, 0)
        pltpu.make_async_copy(stage_hbm.at[pl.ds(0, CH * D)], hrow,
                              sem_h).wait()
        pltpu.make_async_copy(stage_hbm.at[pl.ds(0, CH * D)], trow,
                              sem_t).wait()
        pltpu.make_async_copy(rel_hbm.at[pl.ds(0, CH)], rrow, sem_r).wait()

        def body(g, c2):
            packed = jnp.zeros((L,), jnp.float32)
            for j in range(L):
                item = g * L + j
                acc = jnp.zeros((L,), jnp.float32)
                for c in range(D // L):
                    sl = pl.ds(c * L, L)
                    fsl = pl.ds(item * D + c * L, L)
                    dv = hrow[fsl] + rrow[item, sl] - trow[fsl]
                    acc = acc + dv * dv
                packed = jnp.where(lanes == j, jnp.sum(acc), packed)
            outv[pl.ds(ci * CH + g * L, L)] = _neg_sqrt(packed)
            return c2

        lax.fori_loop(0, CH // L, body, 0)
        return carry

    lax.fori_loop(0, BPW // CH, chunk, 0)
    pltpu.sync_copy(outv, out_hbm.at[pl.ds(base, BPW)])


def kernel(entity_emb, relation_emb, heads, relations, tails):
    heads = heads.astype(jnp.int32)
    relations = relations.astype(jnp.int32)
    tails = tails.astype(jnp.int32)
    stage = _scan_extract(entity_emb.T, heads, tails)
    return _compute(stage, relation_emb, relations)
